# baseline (speedup 1.0000x reference)
import functools

import jax
import jax.numpy as jnp
from jax import lax
from jax.experimental import pallas as pl
from jax.experimental.pallas import tpu as pltpu
from jax.experimental.pallas import tpu_sc as plsc

_B = 16384
_D = 64
_K = 4
_ROW = _K * _D
_NW = 32
_BPW = _B // _NW
_CH = 128
_NCH = _BPW // _CH
_G = 16
_NG = _CH // _G

_mesh = plsc.VectorSubcoreMesh(core_axis_name="c", subcore_axis_name="s")


@functools.partial(
    pl.kernel,
    out_type=jax.ShapeDtypeStruct((_B // _G, _G), jnp.float32),
    mesh=_mesh,
    compiler_params=pltpu.CompilerParams(
        needs_layout_passes=False, use_tc_tiling_on_sc=False),
    scratch_types=[
        pltpu.VMEM((_BPW,), jnp.int32),
        pltpu.VMEM((_BPW, _D), jnp.float32),
        pltpu.VMEM((2, _CH, _ROW), jnp.float32),
        pltpu.VMEM((_BPW // _G, _G), jnp.float32),
        pltpu.SemaphoreType.DMA,
        pltpu.SemaphoreType.DMA,
    ],
)
def _sc_codebook(codes_hbm, pred_hbm, cent_hbm, out_hbm,
                 idx_v, codes_v, rows_v, out_v, sem0, sem1):
    wid = lax.axis_index("s") * 2 + lax.axis_index("c")
    base = pl.multiple_of(wid * _BPW, _BPW)
    gbase = pl.multiple_of(wid * (_BPW // _G), _BPW // _G)

    pltpu.sync_copy(pred_hbm.at[pl.ds(base, _BPW)], idx_v)
    pltpu.sync_copy(codes_hbm.at[pl.ds(base, _BPW)], codes_v)

    sems = (sem0, sem1)

    def start_gather(ch):
        cp = pltpu.make_async_copy(
            cent_hbm.at[idx_v.at[pl.ds(ch * _CH, _CH)]],
            rows_v.at[ch % 2],
            sems[ch % 2],
        )
        cp.start()
        return cp

    lane = lax.iota(jnp.int32, 16)
    zeros16 = jnp.zeros((16,), jnp.int32)
    inv_d = jnp.float32(1.0 / _D)

    def compute_chunk(ch):
        rows = rows_v.at[ch % 2]

        def group_body(g, _):
            rel = g * _G + lane
            rowbase = rel * _ROW
            cbase = (ch * _CH + rel) * _D
            accs = [jnp.zeros((16,), jnp.float32) for _ in range(_K)]
            for d in range(_D):
                code_d = plsc.load_gather(codes_v, [zeros16, cbase + d])
                for k in range(_K):
                    cent = plsc.load_gather(
                        rows, [zeros16, rowbase + (k * _D + d)])
                    accs[k] = accs[k] + jnp.abs(code_d - cent)
            m = jnp.minimum(jnp.minimum(accs[0], accs[1]),
                            jnp.minimum(accs[2], accs[3]))
            gout = (ch * _CH // _G) + g
            plsc.store_scatter(
                out_v, [jnp.full((16,), gout, jnp.int32), lane], m * inv_d)
            return 0

        lax.fori_loop(0, _NG, group_body, 0)

    descs = [None, None]
    descs[0] = start_gather(0)
    for ch in range(_NCH):
        if ch + 1 < _NCH:
            descs[(ch + 1) % 2] = start_gather(ch + 1)
        descs[ch % 2].wait()
        compute_chunk(ch)

    pltpu.sync_copy(out_v, out_hbm.at[pl.ds(gbase, _BPW // _G)])


def kernel(codes, pred_class, centroids):
    cent2d = centroids.reshape(centroids.shape[0], _ROW)
    pred = pred_class.astype(jnp.int32)
    out2d = _sc_codebook(codes, pred, cent2d)
    return out2d.reshape(_B)
